# baseline (speedup 1.0000x reference)
import jax
import jax.numpy as jnp
import numpy as np
from jax.experimental import pallas as pl
from jax.experimental.pallas import tpu as pltpu

N = 10000
E = 320000
NFEAT = 128
NLABEL = 4
NHID = 16
NHEAD = 8
HID = NHEAD * NHID
FDIM = HID + NLABEL


def _sinpos(t, num_steps, dim=128):
    x = t / num_steps * num_steps * 4.0
    half = dim // 2
    emb = jnp.exp(jnp.arange(half, dtype=jnp.float32) * (-(np.log(10000.0) / (half - 1))))
    emb = x[:, None] * emb[None, :]
    return jnp.concatenate([jnp.sin(emb), jnp.cos(emb)], axis=-1)


def _gat(h_in, src, dst, W, asrc, adst, b, n):
    h = (h_in @ W).reshape(n, NHEAD, NHID)
    als = jnp.sum(h * asrc[None, :, :], axis=-1)
    ald = jnp.sum(h * adst[None, :, :], axis=-1)
    e = jax.nn.leaky_relu(als[src] + ald[dst], negative_slope=0.2)
    m = jax.ops.segment_max(e, dst, num_segments=n)
    m = jax.lax.stop_gradient(jnp.where(jnp.isfinite(m), m, 0.0))
    ee = jnp.exp(e - m[dst])
    den = jax.ops.segment_sum(ee, dst, num_segments=n)
    w = ee / (den[dst] + 1e-16)
    out = jax.ops.segment_sum(h[src] * w[:, :, None], dst, num_segments=n)
    return out.reshape(n, NHEAD * NHID) + b


def _elu(x):
    return jnp.where(x > 0, x, jnp.exp(jnp.minimum(x, 0.0)) - 1.0)


def _final_mlp_body(h_ref, fw1_ref, fb1_ref, fw2_ref, fb2_ref, out_ref):
    h = h_ref[...]
    z = jnp.dot(h, fw1_ref[...], preferred_element_type=jnp.float32) + fb1_ref[...]
    z = _elu(z)
    out_ref[...] = jnp.dot(z, fw2_ref[...], preferred_element_type=jnp.float32) + fb2_ref[...]


def _final_mlp(h, fw1, fb1, fw2, fb2):
    return pl.pallas_call(
        _final_mlp_body,
        out_shape=jax.ShapeDtypeStruct((N, NLABEL), jnp.float32),
    )(h, fw1, fb1[None, :], fw2, fb2[None, :])


def kernel(x, q_Y_sample, adj, t, num_steps, W0, asrc0, adst0, b0, W1, asrc1, adst1, b1, tw1, tb1, tw2, tb2, fw1, fb1, fw2, fb2):
    temb = _sinpos(t, num_steps, 128)
    temb = jax.nn.elu(temb @ tw1 + tb1) @ tw2 + tb2
    loop = jnp.arange(N, dtype=adj.dtype)
    src = jnp.concatenate([adj[0], loop])
    dst = jnp.concatenate([adj[1], loop])
    h = jnp.concatenate([x, q_Y_sample], axis=-1)
    h = jax.nn.elu(_gat(h, src, dst, W0, asrc0, adst0, b0, N) + temb)
    h = jnp.concatenate([h, q_Y_sample], axis=-1)
    h = jax.nn.elu(_gat(h, src, dst, W1, asrc1, adst1, b1, N) + temb)
    h = jnp.concatenate([h, q_Y_sample], axis=-1)
    return _final_mlp(h, fw1, fb1, fw2, fb2)


# speedup vs baseline: 42.6506x; 42.6506x over previous
import functools

import jax
import jax.numpy as jnp
import numpy as np
from jax import lax
from jax.experimental import pallas as pl
from jax.experimental.pallas import tpu as pltpu
from jax.experimental.pallas import tpu_sc as plsc

N = 10000
E = 320000
NFEAT = 128
NLABEL = 4
NHID = 16
NHEAD = 8
HID = NHEAD * NHID
FDIM = HID + NLABEL

NWORKERS = 32
CHUNK = 128
E_TOT = E + N
KCHUNKS = 81
EP = NWORKERS * KCHUNKS * CHUNK
N_PAD = 10112
PAD_DST = 10008
ROWS_PER_TILE = 632
ACC_W = 136


def _elu(x):
    return jnp.where(x > 0, x, jnp.exp(jnp.minimum(x, 0.0)) - 1.0)


def _lrelu(x):
    return jnp.maximum(x, 0.2 * x)


def _prep_body(x_ref, qy_ref, t_ref, emb_ref, tw1_ref, tb1_ref, tw2_ref,
               tb2_ref, w0a_ref, w0b_ref, asrc2_ref, adst_ref,
               hp_ref, srcT_ref, dstT_ref, temb_ref):
    ang = (t_ref[...] * 4.0) * emb_ref[...]
    sp = jnp.concatenate([jnp.sin(ang), jnp.cos(ang)], axis=1)
    z = _elu(jnp.dot(sp, tw1_ref[...], preferred_element_type=jnp.float32)
             + tb1_ref[...])
    temb_ref[...] = jnp.dot(z, tw2_ref[...],
                            preferred_element_type=jnp.float32) + tb2_ref[...]

    hp = (jnp.dot(x_ref[...], w0a_ref[...], preferred_element_type=jnp.float32)
          + jnp.dot(qy_ref[...], w0b_ref[...],
                    preferred_element_type=jnp.float32))
    hp_ref[...] = hp
    srcT = jnp.dot(hp, asrc2_ref[...], preferred_element_type=jnp.float32)
    srcT_ref[...] = srcT
    als8 = srcT[:, 0:8]
    ald8 = jnp.dot(hp, adst_ref[...], preferred_element_type=jnp.float32)
    gmax = jnp.max(als8, axis=0, keepdims=True)
    em = jnp.exp(-_lrelu(gmax + ald8))
    dstT_ref[...] = jnp.concatenate([ald8, ald8, em, em], axis=1)


def _prep_call(x, qy, t2d, embc, tw1, tb1, tw2, tb2, w0a, w0b, asrc2, adst):
    return pl.pallas_call(
        _prep_body,
        out_shape=[
            jax.ShapeDtypeStruct((N, HID), jnp.float32),
            jax.ShapeDtypeStruct((N, 16), jnp.float32),
            jax.ShapeDtypeStruct((N, 32), jnp.float32),
            jax.ShapeDtypeStruct((N, HID), jnp.float32),
        ],
    )(x, qy, t2d, embc, tw1, tb1, tw2, tb2, w0a, w0b, asrc2, adst)


def _mid_body(g0_ref, g1_ref, d0_ref, d1_ref, temb_ref, b0_ref, qy_ref,
              mexp_ref, w1a_ref, w1b_ref, asrc2_ref, adst_ref,
              hp_ref, srcT_ref, dstT_ref):
    den = d0_ref[...] + d1_ref[...]
    r = 1.0 / (den + 1e-16)
    rex = jnp.dot(r, mexp_ref[...], preferred_element_type=jnp.float32)
    gat = (g0_ref[...] + g1_ref[...]) * rex
    h = _elu(gat + b0_ref[...] + temb_ref[...])
    hp = (jnp.dot(h, w1a_ref[...], preferred_element_type=jnp.float32)
          + jnp.dot(qy_ref[...], w1b_ref[...],
                    preferred_element_type=jnp.float32))
    hp_ref[...] = hp
    srcT = jnp.dot(hp, asrc2_ref[...], preferred_element_type=jnp.float32)
    srcT_ref[...] = srcT
    als8 = srcT[:, 0:8]
    ald8 = jnp.dot(hp, adst_ref[...], preferred_element_type=jnp.float32)
    gmax = jnp.max(als8, axis=0, keepdims=True)
    em = jnp.exp(-_lrelu(gmax + ald8))
    dstT_ref[...] = jnp.concatenate([ald8, ald8, em, em], axis=1)


def _mid_call(g0, g1, d0, d1, temb, b0, qy, mexp, w1a, w1b, asrc2, adst):
    return pl.pallas_call(
        _mid_body,
        out_shape=[
            jax.ShapeDtypeStruct((N, HID), jnp.float32),
            jax.ShapeDtypeStruct((N, 16), jnp.float32),
            jax.ShapeDtypeStruct((N, 32), jnp.float32),
        ],
    )(g0, g1, d0, d1, temb, b0, qy, mexp, w1a, w1b, asrc2, adst)


def _final_body(g0_ref, g1_ref, d0_ref, d1_ref, temb_ref, b1_ref, qy_ref,
                mexp_ref, fw1a_ref, fw1b_ref, fb1_ref, fw2_ref, fb2_ref,
                out_ref):
    den = d0_ref[...] + d1_ref[...]
    r = 1.0 / (den + 1e-16)
    rex = jnp.dot(r, mexp_ref[...], preferred_element_type=jnp.float32)
    gat = (g0_ref[...] + g1_ref[...]) * rex
    h = _elu(gat + b1_ref[...] + temb_ref[...])
    z = _elu(jnp.dot(h, fw1a_ref[...], preferred_element_type=jnp.float32)
             + jnp.dot(qy_ref[...], fw1b_ref[...],
                       preferred_element_type=jnp.float32)
             + fb1_ref[...])
    out_ref[...] = jnp.dot(z, fw2_ref[...],
                           preferred_element_type=jnp.float32) + fb2_ref[...]


def _final_call(g0, g1, d0, d1, temb, b1, qy, mexp, fw1a, fw1b, fb1, fw2, fb2):
    return pl.pallas_call(
        _final_body,
        out_shape=jax.ShapeDtypeStruct((N, NLABEL), jnp.float32),
    )(g0, g1, d0, d1, temb, b1, qy, mexp, fw1a, fw1b, fb1, fw2, fb2)


@functools.cache
def _sc_edge_kernel_factory():
    mesh = plsc.VectorSubcoreMesh(core_axis_name="c", subcore_axis_name="s")
    return functools.partial(
        pl.kernel,
        out_type=jax.ShapeDtypeStruct((2, N_PAD, ACC_W), jnp.float32),
        mesh=mesh,
        scratch_types=[
            pltpu.VMEM((CHUNK,), jnp.int32),
            pltpu.VMEM((CHUNK,), jnp.int32),
            pltpu.VMEM((CHUNK, 16), jnp.float32),
            pltpu.VMEM((CHUNK, 32), jnp.float32),
            pltpu.VMEM((CHUNK, HID), jnp.float32),
            pltpu.VMEM((CHUNK, ACC_W), jnp.float32),
            pltpu.VMEM_SHARED((N_PAD, ACC_W), jnp.float32),
            pltpu.SemaphoreType.DMA,
            pltpu.SemaphoreType.DMA,
            pltpu.SemaphoreType.DMA,
        ],
        compiler_params=pltpu.CompilerParams(use_tc_tiling_on_sc=False,
                                             needs_layout_passes=False),
    )(_sc_edge_body)


def _sc_edge_kernel(src3d, dst3d, srcT, dstTp, hp, zrows):
    return _sc_edge_kernel_factory()(src3d, dst3d, srcT, dstTp, hp, zrows)


def _sc_edge_body(src_hbm, dst_hbm, srcT_hbm, dstT_hbm, h_hbm, zrows_hbm,
                  out_hbm, srcIc, dstIc, src_b, dst_b, h_b, stage, acc,
                  sem1, sem2, sem3):
    cid = lax.axis_index("c")
    sid = lax.axis_index("s")
    wid = sid * 2 + cid
    base = sid * ROWS_PER_TILE

    pltpu.sync_copy(zrows_hbm, acc.at[pl.ds(base, ROWS_PER_TILE)])
    plsc.subcore_barrier()

    lane = lax.iota(jnp.int32, 16)

    def chunk(k, carry):
        pltpu.sync_copy(src_hbm.at[wid, k], srcIc)
        pltpu.sync_copy(dst_hbm.at[wid, k], dstIc)
        ga = pltpu.async_copy(srcT_hbm.at[srcIc], src_b, sem1)
        gb = pltpu.async_copy(dstT_hbm.at[dstIc], dst_b, sem2)
        gc = pltpu.async_copy(h_hbm.at[srcIc], h_b, sem3)
        ga.wait()
        gb.wait()
        gc.wait()

        def edge(e, c2):
            va = src_b[e, :]
            vd = dst_b[e, pl.ds(0, 16)]
            vm = dst_b[e, pl.ds(16, 16)]
            z = va + vd
            ee = jnp.exp(_lrelu(z)) * vm
            plsc.store_scatter(stage, [jnp.full((16,), e, jnp.int32),
                                       128 + lane], ee, mask=lane < 8)
            for j in range(NHEAD):
                s = ee[j]
                stage[e, pl.ds(16 * j, 16)] = h_b[e, pl.ds(16 * j, 16)] * s
            return c2

        lax.fori_loop(0, CHUNK, edge, 0)
        pltpu.sync_copy(stage, acc.at[dstIc], add=True)
        return carry

    lax.fori_loop(0, KCHUNKS, chunk, 0)
    plsc.subcore_barrier()

    pltpu.sync_copy(acc.at[pl.ds(base, ROWS_PER_TILE)],
                    out_hbm.at[cid, pl.ds(base, ROWS_PER_TILE)])


def _head_mat(a):
    idx = jnp.arange(HID, dtype=jnp.int32)
    return jnp.zeros((HID, NHEAD), jnp.float32).at[idx, idx // NHID].set(
        a.reshape(HID))


def kernel(x, q_Y_sample, adj, t, num_steps, W0, asrc0, adst0, b0,
           W1, asrc1, adst1, b1, tw1, tb1, tw2, tb2, fw1, fb1, fw2, fb2):
    f32 = jnp.float32
    loop = jnp.arange(N, dtype=adj.dtype)
    src = jnp.concatenate([adj[0], loop,
                           jnp.zeros((EP - E_TOT,), adj.dtype)])
    dst = jnp.concatenate([adj[1], loop,
                           jnp.full((EP - E_TOT,), PAD_DST, adj.dtype)])
    src2d = src.reshape(NWORKERS, KCHUNKS, CHUNK)
    dst2d = dst.reshape(NWORKERS, KCHUNKS, CHUNK)

    emb_half = jnp.exp(jnp.arange(64, dtype=f32) * (-(np.log(10000.0) / 63.0)))
    asrcM0 = _head_mat(asrc0)
    asrc2_0 = jnp.concatenate([asrcM0, asrcM0], axis=1)
    adstM0 = _head_mat(adst0)
    asrcM1 = _head_mat(asrc1)
    asrc2_1 = jnp.concatenate([asrcM1, asrcM1], axis=1)
    adstM1 = _head_mat(adst1)
    mexp = (jnp.arange(HID, dtype=jnp.int32)[None, :] // NHID
            == jnp.arange(NHEAD, dtype=jnp.int32)[:, None]).astype(f32)

    hp0, srcT0, dstT0, temb = _prep_call(
        x, q_Y_sample, (t / num_steps * num_steps)[:, None], emb_half[None, :],
        tw1, tb1[None, :], tw2, tb2[None, :],
        W0[:NFEAT], W0[NFEAT:], asrc2_0, adstM0)

    zrows = jnp.zeros((ROWS_PER_TILE, ACC_W), f32)
    dstT0p = jnp.pad(dstT0, ((0, N_PAD - N), (0, 0)))
    acc0 = _sc_edge_kernel(src2d, dst2d, srcT0, dstT0p, hp0, zrows)
    g00 = acc0[0, :N, 0:HID]
    g01 = acc0[1, :N, 0:HID]
    d00 = acc0[0, :N, HID:HID + NHEAD]
    d01 = acc0[1, :N, HID:HID + NHEAD]

    hp1, srcT1, dstT1 = _mid_call(
        g00, g01, d00, d01, temb, b0[None, :], q_Y_sample, mexp,
        W1[:HID], W1[HID:], asrc2_1, adstM1)

    dstT1p = jnp.pad(dstT1, ((0, N_PAD - N), (0, 0)))
    acc1 = _sc_edge_kernel(src2d, dst2d, srcT1, dstT1p, hp1, zrows)
    g10 = acc1[0, :N, 0:HID]
    g11 = acc1[1, :N, 0:HID]
    d10 = acc1[0, :N, HID:HID + NHEAD]
    d11 = acc1[1, :N, HID:HID + NHEAD]

    return _final_call(g10, g11, d10, d11, temb, b1[None, :], q_Y_sample,
                       mexp, fw1[:HID], fw1[HID:], fb1[None, :], fw2,
                       fb2[None, :])


# speedup vs baseline: 61.7254x; 1.4472x over previous
import functools

import jax
import jax.numpy as jnp
import numpy as np
from jax import lax
from jax.experimental import pallas as pl
from jax.experimental.pallas import tpu as pltpu
from jax.experimental.pallas import tpu_sc as plsc

N = 10000
E = 320000
NFEAT = 128
NLABEL = 4
NHID = 16
NHEAD = 8
HID = NHEAD * NHID
FDIM = HID + NLABEL

NWORKERS = 32
CHUNK = 64
E_TOT = E + N
KCHUNKS = 162
EP = NWORKERS * KCHUNKS * CHUNK
N_PAD = 10112
PAD_DST = 10008
ROWS_PER_TILE = 632
ACC_W = 136


def _elu(x):
    return jnp.where(x > 0, x, jnp.exp(jnp.minimum(x, 0.0)) - 1.0)


def _lrelu(x):
    return jnp.maximum(x, 0.2 * x)


def _prep_body(x_ref, qy_ref, t_ref, emb_ref, tw1_ref, tb1_ref, tw2_ref,
               tb2_ref, w0a_ref, w0b_ref, asrc2_ref, adst_ref,
               hp_ref, srcT_ref, dstT_ref, temb_ref):
    ang = (t_ref[...] * 4.0) * emb_ref[...]
    sp = jnp.concatenate([jnp.sin(ang), jnp.cos(ang)], axis=1)
    z = _elu(jnp.dot(sp, tw1_ref[...], preferred_element_type=jnp.float32)
             + tb1_ref[...])
    temb_ref[...] = jnp.dot(z, tw2_ref[...],
                            preferred_element_type=jnp.float32) + tb2_ref[...]

    hp = (jnp.dot(x_ref[...], w0a_ref[...], preferred_element_type=jnp.float32)
          + jnp.dot(qy_ref[...], w0b_ref[...],
                    preferred_element_type=jnp.float32))
    hp_ref[...] = hp
    srcT = jnp.dot(hp, asrc2_ref[...], preferred_element_type=jnp.float32)
    srcT_ref[...] = srcT
    als8 = srcT[:, 0:8]
    ald8 = jnp.dot(hp, adst_ref[...], preferred_element_type=jnp.float32)
    gmax = jnp.max(als8, axis=0, keepdims=True)
    em = jnp.exp(-_lrelu(gmax + ald8))
    dstT_ref[...] = jnp.concatenate([ald8, ald8, em, em], axis=1)


def _prep_call(x, qy, t2d, embc, tw1, tb1, tw2, tb2, w0a, w0b, asrc2, adst):
    return pl.pallas_call(
        _prep_body,
        out_shape=[
            jax.ShapeDtypeStruct((N, HID), jnp.float32),
            jax.ShapeDtypeStruct((N, 16), jnp.float32),
            jax.ShapeDtypeStruct((N, 32), jnp.float32),
            jax.ShapeDtypeStruct((N, HID), jnp.float32),
        ],
    )(x, qy, t2d, embc, tw1, tb1, tw2, tb2, w0a, w0b, asrc2, adst)


def _mid_body(g0_ref, g1_ref, d0_ref, d1_ref, temb_ref, b0_ref, qy_ref,
              mexp_ref, w1a_ref, w1b_ref, asrc2_ref, adst_ref,
              hp_ref, srcT_ref, dstT_ref):
    den = d0_ref[...] + d1_ref[...]
    r = 1.0 / (den + 1e-16)
    rex = jnp.dot(r, mexp_ref[...], preferred_element_type=jnp.float32)
    gat = (g0_ref[...] + g1_ref[...]) * rex
    h = _elu(gat + b0_ref[...] + temb_ref[...])
    hp = (jnp.dot(h, w1a_ref[...], preferred_element_type=jnp.float32)
          + jnp.dot(qy_ref[...], w1b_ref[...],
                    preferred_element_type=jnp.float32))
    hp_ref[...] = hp
    srcT = jnp.dot(hp, asrc2_ref[...], preferred_element_type=jnp.float32)
    srcT_ref[...] = srcT
    als8 = srcT[:, 0:8]
    ald8 = jnp.dot(hp, adst_ref[...], preferred_element_type=jnp.float32)
    gmax = jnp.max(als8, axis=0, keepdims=True)
    em = jnp.exp(-_lrelu(gmax + ald8))
    dstT_ref[...] = jnp.concatenate([ald8, ald8, em, em], axis=1)


def _mid_call(g0, g1, d0, d1, temb, b0, qy, mexp, w1a, w1b, asrc2, adst):
    return pl.pallas_call(
        _mid_body,
        out_shape=[
            jax.ShapeDtypeStruct((N, HID), jnp.float32),
            jax.ShapeDtypeStruct((N, 16), jnp.float32),
            jax.ShapeDtypeStruct((N, 32), jnp.float32),
        ],
    )(g0, g1, d0, d1, temb, b0, qy, mexp, w1a, w1b, asrc2, adst)


def _final_body(g0_ref, g1_ref, d0_ref, d1_ref, temb_ref, b1_ref, qy_ref,
                mexp_ref, fw1a_ref, fw1b_ref, fb1_ref, fw2_ref, fb2_ref,
                out_ref):
    den = d0_ref[...] + d1_ref[...]
    r = 1.0 / (den + 1e-16)
    rex = jnp.dot(r, mexp_ref[...], preferred_element_type=jnp.float32)
    gat = (g0_ref[...] + g1_ref[...]) * rex
    h = _elu(gat + b1_ref[...] + temb_ref[...])
    z = _elu(jnp.dot(h, fw1a_ref[...], preferred_element_type=jnp.float32)
             + jnp.dot(qy_ref[...], fw1b_ref[...],
                       preferred_element_type=jnp.float32)
             + fb1_ref[...])
    out_ref[...] = jnp.dot(z, fw2_ref[...],
                           preferred_element_type=jnp.float32) + fb2_ref[...]


def _final_call(g0, g1, d0, d1, temb, b1, qy, mexp, fw1a, fw1b, fb1, fw2, fb2):
    return pl.pallas_call(
        _final_body,
        out_shape=jax.ShapeDtypeStruct((N, NLABEL), jnp.float32),
    )(g0, g1, d0, d1, temb, b1, qy, mexp, fw1a, fw1b, fb1, fw2, fb2)


@functools.cache
def _sc_edge_kernel_factory():
    mesh = plsc.VectorSubcoreMesh(core_axis_name="c", subcore_axis_name="s")
    return functools.partial(
        pl.kernel,
        out_type=jax.ShapeDtypeStruct((2, N_PAD, ACC_W), jnp.float32),
        mesh=mesh,
        scratch_types=[
            pltpu.VMEM((2, CHUNK), jnp.int32),
            pltpu.VMEM((2, CHUNK), jnp.int32),
            pltpu.VMEM((2, CHUNK), jnp.int32),
            pltpu.VMEM((2, CHUNK, 16), jnp.float32),
            pltpu.VMEM((2, CHUNK, 32), jnp.float32),
            pltpu.VMEM((2, CHUNK, HID), jnp.float32),
            pltpu.VMEM((2, CHUNK, ACC_W), jnp.float32),
            pltpu.VMEM_SHARED((N_PAD, ACC_W), jnp.float32),
            pltpu.SemaphoreType.DMA,
            pltpu.SemaphoreType.DMA,
            pltpu.SemaphoreType.DMA,
            pltpu.SemaphoreType.DMA,
            pltpu.SemaphoreType.DMA,
            pltpu.SemaphoreType.DMA,
        ],
        compiler_params=pltpu.CompilerParams(use_tc_tiling_on_sc=False,
                                             needs_layout_passes=False),
    )(_sc_edge_body)


def _sc_edge_kernel(src3d, dst3d, srcT, dstTp, hp, zrows):
    return _sc_edge_kernel_factory()(src3d, dst3d, srcT, dstTp, hp, zrows)


def _sc_edge_body(src_hbm, dst_hbm, srcT_hbm, dstT_hbm, h_hbm, zrows_hbm,
                  out_hbm, srcIc, dstIc, dstS, src_b, dst_b, h_b, stage, acc,
                  sem_i0, sem_i1, sem_g0, sem_g1, sem_s0, sem_s1):
    cid = lax.axis_index("c")
    sid = lax.axis_index("s")
    wid = sid * 2 + cid
    base = sid * ROWS_PER_TILE
    sem_i = (sem_i0, sem_i1)
    sem_g = (sem_g0, sem_g1)
    sem_s = (sem_s0, sem_s1)

    zc = pltpu.async_copy(zrows_hbm, acc.at[pl.ds(base, ROWS_PER_TILE)],
                          sem_s0)

    lane = lax.iota(jnp.int32, 16)

    def issue_idx(slot, k):
        pltpu.async_copy(src_hbm.at[wid, k], srcIc.at[slot], sem_i[slot])
        pltpu.async_copy(dst_hbm.at[wid, k], dstIc.at[slot], sem_i[slot])

    def wait_idx(slot):
        pltpu.make_async_copy(src_hbm.at[wid, 0], srcIc.at[slot],
                              sem_i[slot]).wait()
        pltpu.make_async_copy(src_hbm.at[wid, 0], dstIc.at[slot],
                              sem_i[slot]).wait()

    def issue_gathers(slot):
        pltpu.async_copy(srcT_hbm.at[srcIc.at[slot]], src_b.at[slot],
                         sem_g[slot])
        pltpu.async_copy(dstT_hbm.at[dstIc.at[slot]], dst_b.at[slot],
                         sem_g[slot])
        pltpu.async_copy(h_hbm.at[srcIc.at[slot]], h_b.at[slot], sem_g[slot])

    def wait_gathers(slot):
        pltpu.make_async_copy(srcT_hbm.at[srcIc.at[slot]], src_b.at[slot],
                              sem_g[slot]).wait()
        pltpu.make_async_copy(dstT_hbm.at[dstIc.at[slot]], dst_b.at[slot],
                              sem_g[slot]).wait()
        pltpu.make_async_copy(h_hbm.at[srcIc.at[slot]], h_b.at[slot],
                              sem_g[slot]).wait()

    def wait_scatter(slot):
        pltpu.make_async_copy(stage.at[slot], acc.at[dstS.at[slot]],
                              sem_s[slot]).wait()

    def compute(slot):
        def edge(e, c2):
            va = src_b[slot, e, :]
            vd = dst_b[slot, e, pl.ds(0, 16)]
            vm = dst_b[slot, e, pl.ds(16, 16)]
            z = va + vd
            ee = jnp.exp(_lrelu(z)) * vm
            plsc.store_scatter(stage.at[slot],
                               [jnp.full((16,), e, jnp.int32), 128 + lane],
                               ee, mask=lane < 8)
            for j in range(NHEAD):
                s = ee[j]
                stage[slot, e, pl.ds(16 * j, 16)] = (
                    h_b[slot, e, pl.ds(16 * j, 16)] * s)
            return c2

        lax.fori_loop(0, CHUNK, edge, 0)

    def half(slot, i, c):
        wait_gathers(slot)

        @pl.when(i > 0)
        def _():
            wait_scatter(slot)
        for q in range(CHUNK // 16):
            dstS[slot, pl.ds(16 * q, 16)] = dstIc[slot, pl.ds(16 * q, 16)]

        @pl.when(i < (KCHUNKS // 2) - 1)
        def _():
            issue_idx(slot, c + 2)
        compute(slot)
        pltpu.async_copy(stage.at[slot], acc.at[dstS.at[slot]], sem_s[slot],
                         add=True)

        @pl.when(i < (KCHUNKS // 2) - 1)
        def _():
            wait_idx(slot)
            issue_gathers(slot)

    issue_idx(0, 0)
    issue_idx(1, 1)
    zc.wait()
    plsc.subcore_barrier()
    wait_idx(0)
    issue_gathers(0)
    wait_idx(1)
    issue_gathers(1)

    def pair(i, carry):
        half(0, i, 2 * i)
        half(1, i, 2 * i + 1)
        return carry

    lax.fori_loop(0, KCHUNKS // 2, pair, 0)
    wait_scatter(0)
    wait_scatter(1)
    plsc.subcore_barrier()

    pltpu.sync_copy(acc.at[pl.ds(base, ROWS_PER_TILE)],
                    out_hbm.at[cid, pl.ds(base, ROWS_PER_TILE)])


def _head_mat(a):
    idx = jnp.arange(HID, dtype=jnp.int32)
    return jnp.zeros((HID, NHEAD), jnp.float32).at[idx, idx // NHID].set(
        a.reshape(HID))


def kernel(x, q_Y_sample, adj, t, num_steps, W0, asrc0, adst0, b0,
           W1, asrc1, adst1, b1, tw1, tb1, tw2, tb2, fw1, fb1, fw2, fb2):
    f32 = jnp.float32
    loop = jnp.arange(N, dtype=adj.dtype)
    src = jnp.concatenate([adj[0], loop,
                           jnp.zeros((EP - E_TOT,), adj.dtype)])
    dst = jnp.concatenate([adj[1], loop,
                           jnp.full((EP - E_TOT,), PAD_DST, adj.dtype)])
    src2d = src.reshape(NWORKERS, KCHUNKS, CHUNK)
    dst2d = dst.reshape(NWORKERS, KCHUNKS, CHUNK)

    emb_half = jnp.exp(jnp.arange(64, dtype=f32) * (-(np.log(10000.0) / 63.0)))
    asrcM0 = _head_mat(asrc0)
    asrc2_0 = jnp.concatenate([asrcM0, asrcM0], axis=1)
    adstM0 = _head_mat(adst0)
    asrcM1 = _head_mat(asrc1)
    asrc2_1 = jnp.concatenate([asrcM1, asrcM1], axis=1)
    adstM1 = _head_mat(adst1)
    mexp = (jnp.arange(HID, dtype=jnp.int32)[None, :] // NHID
            == jnp.arange(NHEAD, dtype=jnp.int32)[:, None]).astype(f32)

    hp0, srcT0, dstT0, temb = _prep_call(
        x, q_Y_sample, (t / num_steps * num_steps)[:, None], emb_half[None, :],
        tw1, tb1[None, :], tw2, tb2[None, :],
        W0[:NFEAT], W0[NFEAT:], asrc2_0, adstM0)

    zrows = jnp.zeros((ROWS_PER_TILE, ACC_W), f32)
    dstT0p = jnp.pad(dstT0, ((0, N_PAD - N), (0, 0)))
    acc0 = _sc_edge_kernel(src2d, dst2d, srcT0, dstT0p, hp0, zrows)
    g00 = acc0[0, :N, 0:HID]
    g01 = acc0[1, :N, 0:HID]
    d00 = acc0[0, :N, HID:HID + NHEAD]
    d01 = acc0[1, :N, HID:HID + NHEAD]

    hp1, srcT1, dstT1 = _mid_call(
        g00, g01, d00, d01, temb, b0[None, :], q_Y_sample, mexp,
        W1[:HID], W1[HID:], asrc2_1, adstM1)

    dstT1p = jnp.pad(dstT1, ((0, N_PAD - N), (0, 0)))
    acc1 = _sc_edge_kernel(src2d, dst2d, srcT1, dstT1p, hp1, zrows)
    g10 = acc1[0, :N, 0:HID]
    g11 = acc1[1, :N, 0:HID]
    d10 = acc1[0, :N, HID:HID + NHEAD]
    d11 = acc1[1, :N, HID:HID + NHEAD]

    return _final_call(g10, g11, d10, d11, temb, b1[None, :], q_Y_sample,
                       mexp, fw1[:HID], fw1[HID:], fb1[None, :], fw2,
                       fb2[None, :])


# speedup vs baseline: 116.8570x; 1.8932x over previous
import functools

import jax
import jax.numpy as jnp
import numpy as np
from jax import lax
from jax.experimental import pallas as pl
from jax.experimental.pallas import tpu as pltpu
from jax.experimental.pallas import tpu_sc as plsc

N = 10000
E = 320000
NFEAT = 128
NLABEL = 4
NHID = 16
NHEAD = 8
HID = NHEAD * NHID
FDIM = HID + NLABEL

NWORKERS = 32
CHUNK = 64
E_TOT = E + N
KCHUNKS = 162
EP = NWORKERS * KCHUNKS * CHUNK
N_PAD = 10112
PAD_DST = 10008
ROWS_PER_TILE = 632
ACC_W = 136


def _elu(x):
    return jnp.where(x > 0, x, jnp.exp(jnp.minimum(x, 0.0)) - 1.0)


def _lrelu(x):
    return jnp.maximum(x, 0.2 * x)


def _prep_body(x_ref, qy_ref, t_ref, emb_ref, tw1_ref, tb1_ref, tw2_ref,
               tb2_ref, w0a_ref, w0b_ref, asrc2_ref, adst_ref,
               hp_ref, srcT_ref, dstT_ref, temb_ref):
    ang = (t_ref[...] * 4.0) * emb_ref[...]
    sp = jnp.concatenate([jnp.sin(ang), jnp.cos(ang)], axis=1)
    z = _elu(jnp.dot(sp, tw1_ref[...], preferred_element_type=jnp.float32)
             + tb1_ref[...])
    temb_ref[...] = jnp.dot(z, tw2_ref[...],
                            preferred_element_type=jnp.float32) + tb2_ref[...]

    hp = (jnp.dot(x_ref[...], w0a_ref[...], preferred_element_type=jnp.float32)
          + jnp.dot(qy_ref[...], w0b_ref[...],
                    preferred_element_type=jnp.float32))
    hp_ref[...] = hp
    srcT = jnp.dot(hp, asrc2_ref[...], preferred_element_type=jnp.float32)
    srcT_ref[...] = srcT
    als8 = srcT[:, 0:8]
    ald8 = jnp.dot(hp, adst_ref[...], preferred_element_type=jnp.float32)
    gmax = jnp.max(als8, axis=0, keepdims=True)
    em = jnp.exp(-_lrelu(gmax + ald8))
    dstT_ref[...] = jnp.concatenate([ald8, ald8, em, em], axis=1)


def _prep_call(x, qy, t2d, embc, tw1, tb1, tw2, tb2, w0a, w0b, asrc2, adst):
    return pl.pallas_call(
        _prep_body,
        out_shape=[
            jax.ShapeDtypeStruct((N, HID), jnp.float32),
            jax.ShapeDtypeStruct((N, 16), jnp.float32),
            jax.ShapeDtypeStruct((N, 32), jnp.float32),
            jax.ShapeDtypeStruct((N, HID), jnp.float32),
        ],
    )(x, qy, t2d, embc, tw1, tb1, tw2, tb2, w0a, w0b, asrc2, adst)


def _mid_body(g0_ref, g1_ref, d0_ref, d1_ref, temb_ref, b0_ref, qy_ref,
              mexp_ref, w1a_ref, w1b_ref, asrc2_ref, adst_ref,
              hp_ref, srcT_ref, dstT_ref):
    den = d0_ref[...] + d1_ref[...]
    r = 1.0 / (den + 1e-16)
    rex = jnp.dot(r, mexp_ref[...], preferred_element_type=jnp.float32)
    gat = (g0_ref[...] + g1_ref[...]) * rex
    h = _elu(gat + b0_ref[...] + temb_ref[...])
    hp = (jnp.dot(h, w1a_ref[...], preferred_element_type=jnp.float32)
          + jnp.dot(qy_ref[...], w1b_ref[...],
                    preferred_element_type=jnp.float32))
    hp_ref[...] = hp
    srcT = jnp.dot(hp, asrc2_ref[...], preferred_element_type=jnp.float32)
    srcT_ref[...] = srcT
    als8 = srcT[:, 0:8]
    ald8 = jnp.dot(hp, adst_ref[...], preferred_element_type=jnp.float32)
    gmax = jnp.max(als8, axis=0, keepdims=True)
    em = jnp.exp(-_lrelu(gmax + ald8))
    dstT_ref[...] = jnp.concatenate([ald8, ald8, em, em], axis=1)


def _mid_call(g0, g1, d0, d1, temb, b0, qy, mexp, w1a, w1b, asrc2, adst):
    return pl.pallas_call(
        _mid_body,
        out_shape=[
            jax.ShapeDtypeStruct((N, HID), jnp.float32),
            jax.ShapeDtypeStruct((N, 16), jnp.float32),
            jax.ShapeDtypeStruct((N, 32), jnp.float32),
        ],
    )(g0, g1, d0, d1, temb, b0, qy, mexp, w1a, w1b, asrc2, adst)


def _final_body(g0_ref, g1_ref, d0_ref, d1_ref, temb_ref, b1_ref, qy_ref,
                mexp_ref, fw1a_ref, fw1b_ref, fb1_ref, fw2_ref, fb2_ref,
                out_ref):
    den = d0_ref[...] + d1_ref[...]
    r = 1.0 / (den + 1e-16)
    rex = jnp.dot(r, mexp_ref[...], preferred_element_type=jnp.float32)
    gat = (g0_ref[...] + g1_ref[...]) * rex
    h = _elu(gat + b1_ref[...] + temb_ref[...])
    z = _elu(jnp.dot(h, fw1a_ref[...], preferred_element_type=jnp.float32)
             + jnp.dot(qy_ref[...], fw1b_ref[...],
                       preferred_element_type=jnp.float32)
             + fb1_ref[...])
    out_ref[...] = jnp.dot(z, fw2_ref[...],
                           preferred_element_type=jnp.float32) + fb2_ref[...]


def _final_call(g0, g1, d0, d1, temb, b1, qy, mexp, fw1a, fw1b, fb1, fw2, fb2):
    return pl.pallas_call(
        _final_body,
        out_shape=jax.ShapeDtypeStruct((N, NLABEL), jnp.float32),
    )(g0, g1, d0, d1, temb, b1, qy, mexp, fw1a, fw1b, fb1, fw2, fb2)


@functools.cache
def _sc_edge_kernel_factory():
    mesh = plsc.VectorSubcoreMesh(core_axis_name="c", subcore_axis_name="s")
    return functools.partial(
        pl.kernel,
        out_type=jax.ShapeDtypeStruct((2, N_PAD, ACC_W), jnp.float32),
        mesh=mesh,
        scratch_types=[
            pltpu.VMEM((2, CHUNK), jnp.int32),
            pltpu.VMEM((2, CHUNK), jnp.int32),
            pltpu.VMEM((2, CHUNK), jnp.int32),
            pltpu.VMEM((2, CHUNK, 16), jnp.float32),
            pltpu.VMEM((2, CHUNK, 32), jnp.float32),
            pltpu.VMEM((2, CHUNK, HID), jnp.float32),
            pltpu.VMEM((2, CHUNK, ACC_W), jnp.float32),
            pltpu.VMEM_SHARED((N_PAD, ACC_W), jnp.float32),
            pltpu.SemaphoreType.DMA,
            pltpu.SemaphoreType.DMA,
            pltpu.SemaphoreType.DMA,
            pltpu.SemaphoreType.DMA,
            pltpu.SemaphoreType.DMA,
            pltpu.SemaphoreType.DMA,
        ],
        compiler_params=pltpu.CompilerParams(use_tc_tiling_on_sc=False,
                                             needs_layout_passes=False),
    )(_sc_edge_body)


def _sc_edge_kernel(src3d, dst3d, srcT, dstTp, hp, zrows):
    return _sc_edge_kernel_factory()(src3d, dst3d, srcT, dstTp, hp, zrows)


def _sc_edge_body(src_hbm, dst_hbm, srcT_hbm, dstT_hbm, h_hbm, zrows_hbm,
                  out_hbm, srcIc, dstIc, dstS, src_b, dst_b, h_b, stage, acc,
                  sem_i0, sem_i1, sem_g0, sem_g1, sem_s0, sem_s1):
    cid = lax.axis_index("c")
    sid = lax.axis_index("s")
    wid = sid * 2 + cid
    base = sid * ROWS_PER_TILE
    sem_i = (sem_i0, sem_i1)
    sem_g = (sem_g0, sem_g1)
    sem_s = (sem_s0, sem_s1)

    zc = pltpu.async_copy(zrows_hbm, acc.at[pl.ds(base, ROWS_PER_TILE)],
                          sem_s0)

    lane = lax.iota(jnp.int32, 16)

    def issue_idx(slot, k):
        pltpu.async_copy(src_hbm.at[wid, k], srcIc.at[slot], sem_i[slot])
        pltpu.async_copy(dst_hbm.at[wid, k], dstIc.at[slot], sem_i[slot])

    def wait_idx(slot):
        pltpu.make_async_copy(src_hbm.at[wid, 0], srcIc.at[slot],
                              sem_i[slot]).wait()
        pltpu.make_async_copy(src_hbm.at[wid, 0], dstIc.at[slot],
                              sem_i[slot]).wait()

    def issue_gathers(slot):
        pltpu.async_copy(srcT_hbm.at[srcIc.at[slot]], src_b.at[slot],
                         sem_g[slot])
        pltpu.async_copy(dstT_hbm.at[dstIc.at[slot]], dst_b.at[slot],
                         sem_g[slot])
        pltpu.async_copy(h_hbm.at[srcIc.at[slot]], h_b.at[slot], sem_g[slot])

    def wait_gathers(slot):
        pltpu.make_async_copy(srcT_hbm.at[srcIc.at[slot]], src_b.at[slot],
                              sem_g[slot]).wait()
        pltpu.make_async_copy(dstT_hbm.at[dstIc.at[slot]], dst_b.at[slot],
                              sem_g[slot]).wait()
        pltpu.make_async_copy(h_hbm.at[srcIc.at[slot]], h_b.at[slot],
                              sem_g[slot]).wait()

    def wait_scatter(slot):
        pltpu.make_async_copy(stage.at[slot], acc.at[dstS.at[slot]],
                              sem_s[slot]).wait()

    lomask = lane < 8
    ecol = 128 + lane

    def compute(slot):
        @plsc.parallel_loop(0, CHUNK, unroll=4)
        def edge(e):
            va = src_b[slot, e, :]
            vd = dst_b[slot, e, pl.ds(0, 16)]
            vm = dst_b[slot, e, pl.ds(16, 16)]
            z = va + vd
            ee = jnp.exp(_lrelu(z)) * vm
            plsc.store_scatter(stage.at[slot],
                               [jnp.full((16,), e, jnp.int32), ecol],
                               ee, mask=lomask)
            for j in range(NHEAD):
                s = ee[j]
                stage[slot, e, pl.ds(16 * j, 16)] = (
                    h_b[slot, e, pl.ds(16 * j, 16)] * s)

    def half(slot, i, c):
        wait_gathers(slot)

        @pl.when(i > 0)
        def _():
            wait_scatter(slot)
        for q in range(CHUNK // 16):
            dstS[slot, pl.ds(16 * q, 16)] = dstIc[slot, pl.ds(16 * q, 16)]

        @pl.when(i < (KCHUNKS // 2) - 1)
        def _():
            issue_idx(slot, c + 2)
        compute(slot)
        pltpu.async_copy(stage.at[slot], acc.at[dstS.at[slot]], sem_s[slot],
                         add=True)

        @pl.when(i < (KCHUNKS // 2) - 1)
        def _():
            wait_idx(slot)
            issue_gathers(slot)

    issue_idx(0, 0)
    issue_idx(1, 1)
    zc.wait()
    plsc.subcore_barrier()
    wait_idx(0)
    issue_gathers(0)
    wait_idx(1)
    issue_gathers(1)

    def pair(i, carry):
        half(0, i, 2 * i)
        half(1, i, 2 * i + 1)
        return carry

    lax.fori_loop(0, KCHUNKS // 2, pair, 0)
    wait_scatter(0)
    wait_scatter(1)
    plsc.subcore_barrier()

    pltpu.sync_copy(acc.at[pl.ds(base, ROWS_PER_TILE)],
                    out_hbm.at[cid, pl.ds(base, ROWS_PER_TILE)])


def _head_mat(a):
    idx = jnp.arange(HID, dtype=jnp.int32)
    return jnp.zeros((HID, NHEAD), jnp.float32).at[idx, idx // NHID].set(
        a.reshape(HID))


def kernel(x, q_Y_sample, adj, t, num_steps, W0, asrc0, adst0, b0,
           W1, asrc1, adst1, b1, tw1, tb1, tw2, tb2, fw1, fb1, fw2, fb2):
    f32 = jnp.float32
    loop = jnp.arange(N, dtype=adj.dtype)
    src = jnp.concatenate([adj[0], loop,
                           jnp.zeros((EP - E_TOT,), adj.dtype)])
    dst = jnp.concatenate([adj[1], loop,
                           jnp.full((EP - E_TOT,), PAD_DST, adj.dtype)])
    src2d = src.reshape(NWORKERS, KCHUNKS, CHUNK)
    dst2d = dst.reshape(NWORKERS, KCHUNKS, CHUNK)

    emb_half = jnp.exp(jnp.arange(64, dtype=f32) * (-(np.log(10000.0) / 63.0)))
    asrcM0 = _head_mat(asrc0)
    asrc2_0 = jnp.concatenate([asrcM0, asrcM0], axis=1)
    adstM0 = _head_mat(adst0)
    asrcM1 = _head_mat(asrc1)
    asrc2_1 = jnp.concatenate([asrcM1, asrcM1], axis=1)
    adstM1 = _head_mat(adst1)
    mexp = (jnp.arange(HID, dtype=jnp.int32)[None, :] // NHID
            == jnp.arange(NHEAD, dtype=jnp.int32)[:, None]).astype(f32)

    hp0, srcT0, dstT0, temb = _prep_call(
        x, q_Y_sample, (t / num_steps * num_steps)[:, None], emb_half[None, :],
        tw1, tb1[None, :], tw2, tb2[None, :],
        W0[:NFEAT], W0[NFEAT:], asrc2_0, adstM0)

    zrows = jnp.zeros((ROWS_PER_TILE, ACC_W), f32)
    dstT0p = jnp.pad(dstT0, ((0, N_PAD - N), (0, 0)))
    acc0 = _sc_edge_kernel(src2d, dst2d, srcT0, dstT0p, hp0, zrows)
    g00 = acc0[0, :N, 0:HID]
    g01 = acc0[1, :N, 0:HID]
    d00 = acc0[0, :N, HID:HID + NHEAD]
    d01 = acc0[1, :N, HID:HID + NHEAD]

    hp1, srcT1, dstT1 = _mid_call(
        g00, g01, d00, d01, temb, b0[None, :], q_Y_sample, mexp,
        W1[:HID], W1[HID:], asrc2_1, adstM1)

    dstT1p = jnp.pad(dstT1, ((0, N_PAD - N), (0, 0)))
    acc1 = _sc_edge_kernel(src2d, dst2d, srcT1, dstT1p, hp1, zrows)
    g10 = acc1[0, :N, 0:HID]
    g11 = acc1[1, :N, 0:HID]
    d10 = acc1[0, :N, HID:HID + NHEAD]
    d11 = acc1[1, :N, HID:HID + NHEAD]

    return _final_call(g10, g11, d10, d11, temb, b1[None, :], q_Y_sample,
                       mexp, fw1[:HID], fw1[HID:], fb1[None, :], fw2,
                       fb2[None, :])
